# baseline (speedup 1.0000x reference)
import functools

import jax
import jax.numpy as jnp
from jax import lax
from jax.experimental import pallas as pl
from jax.experimental.pallas import tpu as pltpu
from jax.experimental.pallas import tpu_sc as plsc

_N = 10000
_DEG = 32
_D = 128
_NC = 2
_NS = 16
_NW = _NC * _NS
_NPW = 320
_NPAD = _NW * _NPW
_CHUNKS = (128, 128, 64)
_OFFS = (0, 128, 256)
_CHMAX = 128
_NCH = len(_CHUNKS)


def _sc_gather_sum(x, nbrt):
    mesh = plsc.VectorSubcoreMesh(core_axis_name="c", subcore_axis_name="s")

    @functools.partial(
        pl.kernel,
        out_type=jax.ShapeDtypeStruct((_NPAD, _D), jnp.float32),
        mesh=mesh,
        scratch_types=[
            pltpu.VMEM((_DEG, _NPW), jnp.int32),
            pltpu.VMEM((_CHMAX, _D), jnp.float32),
            pltpu.VMEM((_CHMAX, _D), jnp.float32),
            pltpu.VMEM_SHARED((_N, _D), jnp.float32),
            pltpu.SemaphoreType.DMA,
            pltpu.SemaphoreType.DMA,
            pltpu.SemaphoreType.DMA,
        ],
    )
    def body(x_hbm, nbrt_hbm, out_hbm, idx_v, acc0, acc1, xs, g0, g1, osem):
        wid = lax.axis_index("s") * _NC + lax.axis_index("c")
        base = wid * _NPW
        sid = lax.axis_index("s")

        @pl.when(sid < _NS - 1)
        def _stage():
            pltpu.sync_copy(x_hbm.at[pl.ds(sid * 624, 624)],
                            xs.at[pl.ds(sid * 624, 624)])

        @pl.when(sid == _NS - 1)
        def _stage_last():
            pltpu.sync_copy(x_hbm.at[pl.ds(15 * 624, _N - 15 * 624)],
                            xs.at[pl.ds(15 * 624, _N - 15 * 624)])
        pltpu.sync_copy(nbrt_hbm.at[:, wid], idx_v)
        plsc.subcore_barrier()

        accs = (acc0, acc1)
        gsems = (g0, g1)
        zero = jnp.zeros((16,), jnp.float32)

        def zero_chunk(buf, rows):
            @pl.loop(0, rows * (_D // 16), unroll=8)
            def _z(i):
                buf[i // (_D // 16), pl.ds((i % (_D // 16)) * 16, 16)] = zero

        def fire(c):
            buf, n = accs[c % 2], _CHUNKS[c]
            dst = buf.at[pl.ds(0, n)] if n != _CHMAX else buf
            return [
                pltpu.async_copy(
                    xs.at[idx_v.at[j, pl.ds(_OFFS[c], n)]], dst,
                    gsems[c % 2], add=True)
                for j in range(_DEG)
            ]

        def writeout(c):
            buf, n = accs[c % 2], _CHUNKS[c]
            src = buf.at[pl.ds(0, n)] if n != _CHMAX else buf
            return pltpu.async_copy(
                src, out_hbm.at[pl.ds(base + _OFFS[c], n)], osem)

        outs = {}
        zero_chunk(accs[0], _CHUNKS[0])
        pend = fire(0)
        for c in range(1, _NCH):
            if c >= 2:
                outs[c - 2].wait()
            zero_chunk(accs[c % 2], _CHUNKS[c])
            nxt = fire(c)
            for cp in pend:
                cp.wait()
            outs[c - 1] = writeout(c - 1)
            pend = nxt
        for cp in pend:
            cp.wait()
        outs[_NCH - 1] = writeout(_NCH - 1)
        outs[_NCH - 2].wait()
        outs[_NCH - 1].wait()

    return body(x, nbrt)


def _tc_pre(x, Wg, Ws, bias):
    br = 1024
    grid = (_N + br - 1) // br

    def body(x_ref, wg_ref, ws_ref, b_ref, o_ref):
        wsum = (wg_ref[...] + ws_ref[...]).astype(jnp.bfloat16)
        a = lax.dot_general(x_ref[...].astype(jnp.bfloat16), wsum,
                            (((1,), (1,)), ((), ())),
                            preferred_element_type=jnp.float32)
        o_ref[...] = a + b_ref[...]

    return pl.pallas_call(
        body,
        grid=(grid,),
        in_specs=[
            pl.BlockSpec((br, _D), lambda i: (i, 0)),
            pl.BlockSpec((_D, _D), lambda i: (0, 0)),
            pl.BlockSpec((_D, _D), lambda i: (0, 0)),
            pl.BlockSpec((1, _D), lambda i: (0, 0)),
        ],
        out_specs=pl.BlockSpec((br, _D), lambda i: (i, 0)),
        out_shape=jax.ShapeDtypeStruct((_N, _D), jnp.float32),
    )(x, Wg, Ws, bias.reshape(1, _D))


def _tc_post(ns, y0, Wl):
    br = 1024
    grid = (_N + br - 1) // br

    def body(ns_ref, y0_ref, wl_ref, o_ref):
        nm = (ns_ref[...] * (1.0 / _DEG)).astype(jnp.bfloat16)
        a = y0_ref[...] + lax.dot_general(
            nm, wl_ref[...].astype(jnp.bfloat16), (((1,), (1,)), ((), ())),
            preferred_element_type=jnp.float32)
        o_ref[...] = jnp.where(a > 0, a, jnp.exp(a) - 1.0)

    return pl.pallas_call(
        body,
        grid=(grid,),
        in_specs=[
            pl.BlockSpec((br, _D), lambda i: (i, 0)),
            pl.BlockSpec((br, _D), lambda i: (i, 0)),
            pl.BlockSpec((_D, _D), lambda i: (0, 0)),
        ],
        out_specs=pl.BlockSpec((br, _D), lambda i: (i, 0)),
        out_shape=jax.ShapeDtypeStruct((_N, _D), jnp.float32),
    )(ns, y0, Wl)


def kernel(x, edge_index, neighbor_flat, Wg, Wl, Ws, bias):
    del edge_index
    nbr = neighbor_flat.astype(jnp.int32).reshape(_N, _DEG).T
    nbrt = jnp.pad(nbr, ((0, 0), (0, _NPAD - _N)))
    nbrt = nbrt.reshape(_DEG, _NW, _NPW)
    ns = _sc_gather_sum(x, nbrt)
    y0 = _tc_pre(x, Wg, Ws, bias)
    return _tc_post(ns, y0, Wl)


# speedup vs baseline: 1.0060x; 1.0060x over previous
import functools

import jax
import jax.numpy as jnp
from jax import lax
from jax.experimental import pallas as pl
from jax.experimental.pallas import tpu as pltpu
from jax.experimental.pallas import tpu_sc as plsc

_N = 10000
_DEG = 32
_D = 128
_NC = 2
_NS = 16
_NW = _NC * _NS
_NPW = 320
_NPAD = _NW * _NPW
_CHUNKS = (128, 128, 64)
_OFFS = (0, 128, 256)
_CHMAX = 128
_NCH = len(_CHUNKS)


def _sc_gather_sum(x, nbrt):
    mesh = plsc.VectorSubcoreMesh(core_axis_name="c", subcore_axis_name="s")

    @functools.partial(
        pl.kernel,
        out_type=jax.ShapeDtypeStruct((_NPAD, _D), jnp.float32),
        mesh=mesh,
        scratch_types=[
            pltpu.VMEM((_DEG, _NPW), jnp.int32),
            pltpu.VMEM((_CHMAX, _D), jnp.float32),
            pltpu.VMEM((_CHMAX, _D), jnp.float32),
            pltpu.VMEM_SHARED((_N, _D), jnp.float32),
            pltpu.SemaphoreType.DMA,
            pltpu.SemaphoreType.DMA,
            pltpu.SemaphoreType.DMA,
        ],
    )
    def body(x_hbm, nbrt_hbm, out_hbm, idx_v, acc0, acc1, xs, g0, g1, osem):
        wid = lax.axis_index("s") * _NC + lax.axis_index("c")
        base = wid * _NPW
        sid = lax.axis_index("s")

        @pl.when(sid < _NS - 1)
        def _stage():
            pltpu.sync_copy(x_hbm.at[pl.ds(sid * 624, 624)],
                            xs.at[pl.ds(sid * 624, 624)])

        @pl.when(sid == _NS - 1)
        def _stage_last():
            pltpu.sync_copy(x_hbm.at[pl.ds(15 * 624, _N - 15 * 624)],
                            xs.at[pl.ds(15 * 624, _N - 15 * 624)])
        pltpu.sync_copy(nbrt_hbm.at[:, wid], idx_v)
        plsc.subcore_barrier()

        accs = (acc0, acc1)
        gsems = (g0, g1)
        zero = jnp.zeros((16,), jnp.float32)

        def zero_chunk(buf, rows):
            @pl.loop(0, rows * (_D // 16), unroll=8)
            def _z(i):
                buf[i // (_D // 16), pl.ds((i % (_D // 16)) * 16, 16)] = zero

        def fire(c):
            buf, n = accs[c % 2], _CHUNKS[c]
            dst = buf.at[pl.ds(0, n)] if n != _CHMAX else buf
            return [
                pltpu.async_copy(
                    xs.at[idx_v.at[j, pl.ds(_OFFS[c], n)]], dst,
                    gsems[c % 2], add=True)
                for j in range(_DEG)
            ]

        def writeout(c):
            buf, n = accs[c % 2], _CHUNKS[c]
            src = buf.at[pl.ds(0, n)] if n != _CHMAX else buf
            return pltpu.async_copy(
                src, out_hbm.at[pl.ds(base + _OFFS[c], n)], osem)

        outs = {}
        zero_chunk(accs[0], _CHUNKS[0])
        pend = fire(0)
        for c in range(1, _NCH):
            if c >= 2:
                outs[c - 2].wait()
            zero_chunk(accs[c % 2], _CHUNKS[c])
            nxt = fire(c)
            for cp in pend:
                cp.wait()
            outs[c - 1] = writeout(c - 1)
            pend = nxt
        for cp in pend:
            cp.wait()
        outs[_NCH - 1] = writeout(_NCH - 1)
        outs[_NCH - 2].wait()
        outs[_NCH - 1].wait()

    return body(x, nbrt)


def _tc_pre(x, Wg, Ws, bias):
    br = 1024
    grid = (_N + br - 1) // br

    def body(x_ref, wg_ref, ws_ref, b_ref, o_ref):
        wsum = (wg_ref[...] + ws_ref[...]).astype(jnp.bfloat16)
        a = lax.dot_general(x_ref[...].astype(jnp.bfloat16), wsum,
                            (((1,), (1,)), ((), ())),
                            preferred_element_type=jnp.float32)
        o_ref[...] = (a + b_ref[...]).astype(jnp.bfloat16)

    return pl.pallas_call(
        body,
        grid=(grid,),
        in_specs=[
            pl.BlockSpec((br, _D), lambda i: (i, 0)),
            pl.BlockSpec((_D, _D), lambda i: (0, 0)),
            pl.BlockSpec((_D, _D), lambda i: (0, 0)),
            pl.BlockSpec((1, _D), lambda i: (0, 0)),
        ],
        out_specs=pl.BlockSpec((br, _D), lambda i: (i, 0)),
        out_shape=jax.ShapeDtypeStruct((_N, _D), jnp.bfloat16),
    )(x, Wg, Ws, bias.reshape(1, _D))


def _tc_post(ns, y0, Wl):
    br = 1024
    grid = (_N + br - 1) // br

    def body(ns_ref, y0_ref, wl_ref, o_ref):
        nm = (ns_ref[...] * (1.0 / _DEG)).astype(jnp.bfloat16)
        a = y0_ref[...] + lax.dot_general(
            nm, wl_ref[...].astype(jnp.bfloat16), (((1,), (1,)), ((), ())),
            preferred_element_type=jnp.float32)
        o_ref[...] = jnp.where(a > 0, a, jnp.exp(a) - 1.0)

    return pl.pallas_call(
        body,
        grid=(grid,),
        in_specs=[
            pl.BlockSpec((br, _D), lambda i: (i, 0)),
            pl.BlockSpec((br, _D), lambda i: (i, 0)),
            pl.BlockSpec((_D, _D), lambda i: (0, 0)),
        ],
        out_specs=pl.BlockSpec((br, _D), lambda i: (i, 0)),
        out_shape=jax.ShapeDtypeStruct((_N, _D), jnp.float32),
    )(ns, y0, Wl)


def kernel(x, edge_index, neighbor_flat, Wg, Wl, Ws, bias):
    del edge_index
    nbr = neighbor_flat.astype(jnp.int32).reshape(_N, _DEG).T
    nbrt = jnp.pad(nbr, ((0, 0), (0, _NPAD - _N)))
    nbrt = nbrt.reshape(_DEG, _NW, _NPW)
    ns = _sc_gather_sum(x, nbrt)
    y0 = _tc_pre(x, Wg, Ws, bias)
    return _tc_post(ns, y0, Wl)
